# baseline (speedup 1.0000x reference)
import functools

import jax
import jax.numpy as jnp
from jax import lax
from jax.experimental import pallas as pl
from jax.experimental.pallas import tpu as pltpu
from jax.experimental.pallas import tpu_sc as plsc

_B = 16384
_TILE = 1024
_GRID = _B // _TILE
_F = 256
_NC = 2
_NS = 16
_BPW = _B // (_NC * _NS)


def _sc_gather2(idx2, big):
    mesh = plsc.VectorSubcoreMesh(core_axis_name="c", subcore_axis_name="s")

    def body(idx_hbm, big_hbm, uo_hbm, mo_hbm, idx_v, rows_v, sem):
        wid = lax.axis_index("s") * _NC + lax.axis_index("c")
        base = wid * _BPW
        pltpu.sync_copy(idx_hbm.at[pl.ds(base, _BPW)], idx_v)
        pltpu.async_copy(big_hbm.at[idx_v], rows_v, sem).wait()
        pltpu.sync_copy(rows_v, uo_hbm.at[pl.ds(base, _BPW)])
        pltpu.sync_copy(idx_hbm.at[pl.ds(_B + base, _BPW)], idx_v)
        pltpu.async_copy(big_hbm.at[idx_v], rows_v, sem).wait()
        pltpu.sync_copy(rows_v, mo_hbm.at[pl.ds(base, _BPW)])

    f = pl.kernel(
        body,
        out_type=(
            jax.ShapeDtypeStruct((_B, 128), jnp.float32),
            jax.ShapeDtypeStruct((_B, 128), jnp.float32),
        ),
        mesh=mesh,
        scratch_types=[
            pltpu.VMEM((_BPW,), jnp.int32),
            pltpu.VMEM((_BPW, 128), jnp.float32),
            pltpu.SemaphoreType.DMA,
        ],
    )
    return f(idx2, big)


def _k1(u_ref, mv_ref, ge_ref, ag_ref, oc_ref, gen_ref, rt_ref, im_ref,
        T_ref, Qw_ref, Pw_ref, GWw_ref, gbw_ref, ugbw_ref,
        EGw_ref, EAw_ref, EOw_ref, W1_ref,
        x_ref, s1_ref, q1_ref):
    i = pl.program_id(0)
    f32 = jnp.float32
    bf16 = jnp.bfloat16
    gvw = jnp.dot(gen_ref[...], GWw_ref[...],
                  preferred_element_type=f32) + gbw_ref[...]
    u = u_ref[:, 0:64]
    tmp = jnp.dot(u.astype(bf16), T_ref[...], preferred_element_type=f32)
    gvr = jnp.dot(gvw.astype(bf16), Qw_ref[...], preferred_element_type=f32)
    cross_w = jnp.dot((tmp * gvr).astype(bf16), Pw_ref[...],
                      preferred_element_type=f32) + ugbw_ref[...]
    ge_oh = (ge_ref[...] == lax.broadcasted_iota(jnp.int32, (_TILE, 8), 1)
             ).astype(f32)
    ag_oh = (ag_ref[...] == lax.broadcasted_iota(jnp.int32, (_TILE, 8), 1)
             ).astype(f32)
    oc_oh = (oc_ref[...] == lax.broadcasted_iota(jnp.int32, (_TILE, 32), 1)
             ).astype(f32)
    emb_w = (jnp.dot(ge_oh, EGw_ref[...], preferred_element_type=f32)
             + jnp.dot(ag_oh, EAw_ref[...], preferred_element_type=f32)
             + jnp.dot(oc_oh, EOw_ref[...], preferred_element_type=f32))
    rt = rt_ref[...]
    rt_hi = rt.astype(bf16).astype(f32)
    rt_lo = rt - rt_hi
    imp = im_ref[...]
    z96 = jnp.zeros((_TILE, 96), f32)
    z29 = jnp.zeros((_TILE, 29), f32)
    rtblock = jnp.concatenate([z96, rt_hi, rt_lo, imp, z29], axis=1)
    reg1 = emb_w + gvw + cross_w + rtblock
    x16 = jnp.concatenate([u, mv_ref[:, 64:128], reg1], axis=1).astype(bf16)
    x_ref[...] = x16
    y1 = jnp.dot(x16, W1_ref[...], preferred_element_type=f32)

    @pl.when(i == 0)
    def _():
        s1_ref[...] = jnp.zeros_like(s1_ref)
        q1_ref[...] = jnp.zeros_like(q1_ref)

    s1_ref[...] += jnp.sum(y1, axis=0, keepdims=True)
    q1_ref[...] += jnp.sum(y1 * y1, axis=0, keepdims=True)


def _bn_scale_shift(s_ref, q_ref, g_ref, be_ref):
    mu = s_ref[...] * (1.0 / _B)
    var = q_ref[...] * (1.0 / _B) - mu * mu
    scale = lax.rsqrt(var + 1e-5) * g_ref[...]
    shift = be_ref[...] - mu * scale
    return scale, shift


def _k2(x_ref, s1_ref, q1_ref, g_ref, be_ref,
        W1_ref, pW1_ref, pb1_ref, W2f_ref,
        h1_ref, y2_ref, s2_ref, q2_ref, w2_scr):
    i = pl.program_id(0)
    f32 = jnp.float32
    bf16 = jnp.bfloat16

    @pl.when(i == 0)
    def _():
        w2_scr[...] = W2f_ref[...].astype(bf16)
        s2_ref[...] = jnp.zeros_like(s2_ref)
        q2_ref[...] = jnp.zeros_like(q2_ref)

    scale, shift = _bn_scale_shift(s1_ref, q1_ref, g_ref, be_ref)
    x = x_ref[...]
    y1 = jnp.dot(x, W1_ref[...], preferred_element_type=f32)
    y1p = jnp.dot(x, pW1_ref[...], preferred_element_type=f32) + pb1_ref[...]
    h1 = jnp.maximum(y1 * scale + shift, 0.0) + y1p
    h1b = h1.astype(bf16)
    h1_ref[...] = h1b
    y2 = jnp.dot(h1b, w2_scr[...], preferred_element_type=f32)
    y2b = y2.astype(bf16)
    y2_ref[...] = y2b
    y2f = y2b.astype(f32)

    s2_ref[...] += jnp.sum(y2f, axis=0, keepdims=True)
    q2_ref[...] += jnp.sum(y2f * y2f, axis=0, keepdims=True)


def _k3(h1_ref, y2_ref, s_ref, q_ref, g_ref, be_ref, W3f_ref, pW3f_ref,
        pb3_ref, ow_ref, y3_ref, lo3_ref, s3_ref, q3_ref, w3_scr, pw3_scr):
    i = pl.program_id(0)
    f32 = jnp.float32
    bf16 = jnp.bfloat16

    @pl.when(i == 0)
    def _():
        w3_scr[...] = W3f_ref[...].astype(bf16)
        pw3_scr[...] = pW3f_ref[...].astype(bf16)
        s3_ref[...] = jnp.zeros_like(s3_ref)
        q3_ref[...] = jnp.zeros_like(q3_ref)

    scale, shift = _bn_scale_shift(s_ref, q_ref, g_ref, be_ref)
    h2 = (jnp.maximum(y2_ref[...].astype(f32) * scale + shift, 0.0)
          + h1_ref[...].astype(f32))
    h2b = h2.astype(bf16)
    y3 = jnp.dot(h2b, w3_scr[...], preferred_element_type=f32)
    y3p = jnp.dot(h2b, pw3_scr[...], preferred_element_type=f32) + pb3_ref[...]
    y3b = y3.astype(bf16)
    y3_ref[...] = y3b
    y3f = y3b.astype(f32)
    lo3_ref[...] = jnp.sum(y3p * ow_ref[...], axis=1, keepdims=True)

    s3_ref[...] += jnp.sum(y3f, axis=0, keepdims=True)
    q3_ref[...] += jnp.sum(y3f * y3f, axis=0, keepdims=True)


def _k4(y3_ref, lo3_ref, s_ref, q_ref, g_ref, be_ref, ow_ref, ob_ref, out_ref):
    scale, shift = _bn_scale_shift(s_ref, q_ref, g_ref, be_ref)
    h3r = jnp.maximum(y3_ref[...].astype(jnp.float32) * scale + shift, 0.0)
    out_ref[...] = (jnp.sum(h3r * ow_ref[...], axis=1, keepdims=True)
                    + lo3_ref[...] + ob_ref[...])


def _const_spec(shape):
    nd = len(shape)
    return pl.BlockSpec(shape, lambda i: (0,) * nd)


def _tile_spec(cols, rows=_TILE):
    return pl.BlockSpec((rows, cols), lambda i: (i, 0))


_SEQ = pltpu.CompilerParams(dimension_semantics=("arbitrary",))


def kernel(user_id, gender, age, occupation, movie_id, genres, rating, implicit,
           emb_user, emb_gender, emb_age, emb_occ, emb_movie,
           genre_W, genre_b, ug_W, ug_b,
           b1_W, b1_b, b1_g, b1_beta, b1_pW, b1_pb,
           b2_W, b2_b, b2_g, b2_beta,
           b3_W, b3_b, b3_g, b3_beta, b3_pW, b3_pb,
           out_W, out_b):
    f32 = jnp.float32
    bf16 = jnp.bfloat16
    idx2 = jnp.concatenate([
        user_id.astype(jnp.int32),
        movie_id.astype(jnp.int32),
    ])

    big = jnp.concatenate([emb_user, emb_movie], axis=1)
    u128, m128 = _sc_gather2(idx2, big)
    ge2 = gender.astype(jnp.int32).reshape(_B, 1)
    ag2 = age.astype(jnp.int32).reshape(_B, 1)
    oc2 = occupation.astype(jnp.int32).reshape(_B, 1)
    EGw = jnp.pad(emb_gender, ((0, 4), (0, 120)))
    EAw = jnp.pad(emb_age, ((0, 0), (8, 112)))
    EOw = jnp.pad(emb_occ, ((0, 0), (16, 96)))

    T = ug_W.reshape(64, 1024).astype(bf16)
    c1024 = jnp.arange(1024)
    Qw = jnp.pad((jnp.arange(32)[:, None] == (c1024[None, :] // 32)
                  ).astype(bf16), ((32, 64), (0, 0)))
    Pw = jnp.pad(((c1024[:, None] % 32) == jnp.arange(32)[None, :]
                  ).astype(bf16), ((0, 0), (64, 32)))
    GWw = jnp.pad(genre_W, ((0, 0), (32, 64)))
    gbw = jnp.pad(genre_b[None, :], ((0, 0), (32, 64)))
    ugbw = jnp.pad(ug_b[None, :], ((0, 0), (64, 32)))

    def permute_w1(W):
        return jnp.concatenate(
            [W[0:64], W[96:160], W[64:96], W[160:192], W[194:226],
             W[192:193], W[192:193], W[193:194],
             jnp.zeros((29, W.shape[1]), f32)], axis=0)

    W1b = permute_w1(b1_W).astype(bf16)
    pW1b = permute_w1(b1_pW).astype(bf16)

    x16, s1, q1 = pl.pallas_call(
        _k1,
        grid=(_GRID,),
        in_specs=[
            _tile_spec(128), _tile_spec(128),
            _tile_spec(1), _tile_spec(1), _tile_spec(1),
            _tile_spec(19), _tile_spec(1), _tile_spec(1),
            _const_spec((64, 1024)), _const_spec((128, 1024)),
            _const_spec((1024, 128)), _const_spec((19, 128)),
            _const_spec((1, 128)), _const_spec((1, 128)),
            _const_spec((8, 128)), _const_spec((8, 128)),
            _const_spec((32, 128)),
            _const_spec((_F, 1024)),
        ],
        out_specs=[
            _tile_spec(_F),
            _const_spec((1, 1024)), _const_spec((1, 1024)),
        ],
        out_shape=[
            jax.ShapeDtypeStruct((_B, _F), bf16),
            jax.ShapeDtypeStruct((1, 1024), f32),
            jax.ShapeDtypeStruct((1, 1024), f32),
        ],
        compiler_params=_SEQ,
    )(u128, m128, ge2, ag2, oc2, genres, rating[:, None], implicit[:, None],
      T, Qw, Pw, GWw, gbw, ugbw, EGw, EAw, EOw, W1b)

    h1, y2, s2, q2 = pl.pallas_call(
        _k2,
        grid=(_GRID,),
        in_specs=[
            _tile_spec(_F),
            _const_spec((1, 1024)), _const_spec((1, 1024)),
            _const_spec((1, 1024)), _const_spec((1, 1024)),
            _const_spec((_F, 1024)), _const_spec((_F, 1024)),
            _const_spec((1, 1024)), _const_spec((1024, 1024)),
        ],
        out_specs=[
            _tile_spec(1024), _tile_spec(1024),
            _const_spec((1, 1024)), _const_spec((1, 1024)),
        ],
        out_shape=[
            jax.ShapeDtypeStruct((_B, 1024), bf16),
            jax.ShapeDtypeStruct((_B, 1024), bf16),
            jax.ShapeDtypeStruct((1, 1024), f32),
            jax.ShapeDtypeStruct((1, 1024), f32),
        ],
        scratch_shapes=[pltpu.VMEM((1024, 1024), bf16)],
        compiler_params=_SEQ,
    )(x16, s1, q1, b1_g.reshape(1, 1024), b1_beta.reshape(1, 1024),
      W1b, pW1b, b1_pb.reshape(1, 1024), b2_W)

    ow = out_W.reshape(1, 512)
    t3 = 1024
    y3, lo3, s3, q3 = pl.pallas_call(
        _k3,
        grid=(_B // t3,),
        in_specs=[
            _tile_spec(1024, t3), _tile_spec(1024, t3),
            _const_spec((1, 1024)), _const_spec((1, 1024)),
            _const_spec((1, 1024)), _const_spec((1, 1024)),
            _const_spec((1024, 512)), _const_spec((1024, 512)),
            _const_spec((1, 512)), _const_spec((1, 512)),
        ],
        out_specs=[
            _tile_spec(512, t3), _tile_spec(1, t3),
            _const_spec((1, 512)), _const_spec((1, 512)),
        ],
        out_shape=[
            jax.ShapeDtypeStruct((_B, 512), bf16),
            jax.ShapeDtypeStruct((_B, 1), f32),
            jax.ShapeDtypeStruct((1, 512), f32),
            jax.ShapeDtypeStruct((1, 512), f32),
        ],
        scratch_shapes=[pltpu.VMEM((1024, 512), bf16),
                        pltpu.VMEM((1024, 512), bf16)],
        compiler_params=_SEQ,
    )(h1, y2, s2, q2, b2_g.reshape(1, 1024), b2_beta.reshape(1, 1024),
      b3_W, b3_pW, b3_pb.reshape(1, 512), ow)

    t4 = 2048
    out = pl.pallas_call(
        _k4,
        grid=(_B // t4,),
        in_specs=[
            _tile_spec(512, t4), _tile_spec(1, t4),
            _const_spec((1, 512)), _const_spec((1, 512)),
            _const_spec((1, 512)), _const_spec((1, 512)),
            _const_spec((1, 512)), _const_spec((1, 1)),
        ],
        out_specs=_tile_spec(1, t4),
        out_shape=jax.ShapeDtypeStruct((_B, 1), f32),
        compiler_params=_SEQ,
    )(y3, lo3, s3, q3, b3_g.reshape(1, 512), b3_beta.reshape(1, 512),
      ow, out_b.reshape(1, 1))

    return out[:, 0]


# speedup vs baseline: 1.0679x; 1.0679x over previous
import functools

import jax
import jax.numpy as jnp
from jax import lax
from jax.experimental import pallas as pl
from jax.experimental.pallas import tpu as pltpu
from jax.experimental.pallas import tpu_sc as plsc

_B = 16384
_TILE = 1024
_GRID = _B // _TILE
_F = 256
_NC = 2
_NS = 16
_BPW = _B // (_NC * _NS)


def _sc_gather3(idx3, big, combo):
    mesh = plsc.VectorSubcoreMesh(core_axis_name="c", subcore_axis_name="s")

    def body(idx_hbm, big_hbm, co_hbm,
             uo_hbm, mo_hbm, co_out_hbm, idx_v, rows_v, sem):
        wid = lax.axis_index("s") * _NC + lax.axis_index("c")
        base = wid * _BPW
        pltpu.sync_copy(idx_hbm.at[pl.ds(base, _BPW)], idx_v)
        pltpu.async_copy(big_hbm.at[idx_v], rows_v, sem).wait()
        pltpu.sync_copy(rows_v, uo_hbm.at[pl.ds(base, _BPW)])
        pltpu.sync_copy(idx_hbm.at[pl.ds(_B + base, _BPW)], idx_v)
        pltpu.async_copy(big_hbm.at[idx_v], rows_v, sem).wait()
        pltpu.sync_copy(rows_v, mo_hbm.at[pl.ds(base, _BPW)])
        pltpu.sync_copy(idx_hbm.at[pl.ds(2 * _B + base, _BPW)], idx_v)
        pltpu.async_copy(co_hbm.at[idx_v], rows_v, sem).wait()
        pltpu.sync_copy(rows_v, co_out_hbm.at[pl.ds(base, _BPW)])

    f = pl.kernel(
        body,
        out_type=(
            jax.ShapeDtypeStruct((_B, 128), jnp.float32),
            jax.ShapeDtypeStruct((_B, 128), jnp.float32),
            jax.ShapeDtypeStruct((_B, 128), jnp.float32),
        ),
        mesh=mesh,
        scratch_types=[
            pltpu.VMEM((_BPW,), jnp.int32),
            pltpu.VMEM((_BPW, 128), jnp.float32),
            pltpu.SemaphoreType.DMA,
        ],
    )
    return f(idx3, big, combo)


def _k1(u_ref, mv_ref, c_ref, gen_ref, rt_ref, im_ref,
        T_ref, Qw_ref, Pw_ref, GWw_ref, gbw_ref, ugbw_ref, W1_ref,
        x_ref, s1_ref, q1_ref):
    i = pl.program_id(0)
    f32 = jnp.float32
    bf16 = jnp.bfloat16
    gvw = jnp.dot(gen_ref[...], GWw_ref[...],
                  preferred_element_type=f32) + gbw_ref[...]
    u = u_ref[:, 0:64]
    tmp = jnp.dot(u.astype(bf16), T_ref[...], preferred_element_type=f32)
    gvr = jnp.dot(gvw.astype(bf16), Qw_ref[...], preferred_element_type=f32)
    cross_w = jnp.dot((tmp * gvr).astype(bf16), Pw_ref[...],
                      preferred_element_type=f32) + ugbw_ref[...]
    rt = rt_ref[...]
    rt_hi = rt.astype(bf16).astype(f32)
    rt_lo = rt - rt_hi
    imp = im_ref[...]
    z96 = jnp.zeros((_TILE, 96), f32)
    z29 = jnp.zeros((_TILE, 29), f32)
    rtblock = jnp.concatenate([z96, rt_hi, rt_lo, imp, z29], axis=1)
    reg1 = c_ref[...] + gvw + cross_w + rtblock
    x16 = jnp.concatenate([u, mv_ref[:, 64:128], reg1], axis=1).astype(bf16)
    x_ref[...] = x16
    y1 = jnp.dot(x16, W1_ref[...], preferred_element_type=f32)

    @pl.when(i == 0)
    def _():
        s1_ref[...] = jnp.zeros_like(s1_ref)
        q1_ref[...] = jnp.zeros_like(q1_ref)

    s1_ref[...] += jnp.sum(y1, axis=0, keepdims=True)
    q1_ref[...] += jnp.sum(y1 * y1, axis=0, keepdims=True)


def _bn_scale_shift(s_ref, q_ref, g_ref, be_ref):
    mu = s_ref[...] * (1.0 / _B)
    var = q_ref[...] * (1.0 / _B) - mu * mu
    scale = lax.rsqrt(var + 1e-5) * g_ref[...]
    shift = be_ref[...] - mu * scale
    return scale, shift


def _k2(x_ref, s1_ref, q1_ref, g_ref, be_ref,
        W1_ref, pW1_ref, pb1_ref, W2f_ref,
        h1_ref, y2_ref, s2_ref, q2_ref, w2_scr):
    i = pl.program_id(0)
    f32 = jnp.float32
    bf16 = jnp.bfloat16

    @pl.when(i == 0)
    def _():
        w2_scr[...] = W2f_ref[...].astype(bf16)
        s2_ref[...] = jnp.zeros_like(s2_ref)
        q2_ref[...] = jnp.zeros_like(q2_ref)

    scale, shift = _bn_scale_shift(s1_ref, q1_ref, g_ref, be_ref)
    x = x_ref[...]
    y1 = jnp.dot(x, W1_ref[...], preferred_element_type=f32)
    y1p = jnp.dot(x, pW1_ref[...], preferred_element_type=f32) + pb1_ref[...]
    h1 = jnp.maximum(y1 * scale + shift, 0.0) + y1p
    h1b = h1.astype(bf16)
    h1_ref[...] = h1b
    y2 = jnp.dot(h1b, w2_scr[...], preferred_element_type=f32)
    y2b = y2.astype(bf16)
    y2_ref[...] = y2b
    y2f = y2b.astype(f32)

    s2_ref[...] += jnp.sum(y2f, axis=0, keepdims=True)
    q2_ref[...] += jnp.sum(y2f * y2f, axis=0, keepdims=True)


def _k3(h1_ref, y2_ref, s_ref, q_ref, g_ref, be_ref, W3f_ref, pW3f_ref,
        pb3_ref, ow_ref, y3_ref, lo3_ref, s3_ref, q3_ref, w3_scr, pw3_scr):
    i = pl.program_id(0)
    f32 = jnp.float32
    bf16 = jnp.bfloat16

    @pl.when(i == 0)
    def _():
        w3_scr[...] = W3f_ref[...].astype(bf16)
        pw3_scr[...] = pW3f_ref[...].astype(bf16)
        s3_ref[...] = jnp.zeros_like(s3_ref)
        q3_ref[...] = jnp.zeros_like(q3_ref)

    scale, shift = _bn_scale_shift(s_ref, q_ref, g_ref, be_ref)
    h2 = (jnp.maximum(y2_ref[...].astype(f32) * scale + shift, 0.0)
          + h1_ref[...].astype(f32))
    h2b = h2.astype(bf16)
    y3 = jnp.dot(h2b, w3_scr[...], preferred_element_type=f32)
    y3p = jnp.dot(h2b, pw3_scr[...], preferred_element_type=f32) + pb3_ref[...]
    y3b = y3.astype(bf16)
    y3_ref[...] = y3b
    y3f = y3b.astype(f32)
    lo3_ref[...] = jnp.sum(y3p * ow_ref[...], axis=1, keepdims=True)

    s3_ref[...] += jnp.sum(y3f, axis=0, keepdims=True)
    q3_ref[...] += jnp.sum(y3f * y3f, axis=0, keepdims=True)


def _k4(y3_ref, lo3_ref, s_ref, q_ref, g_ref, be_ref, ow_ref, ob_ref, out_ref):
    scale, shift = _bn_scale_shift(s_ref, q_ref, g_ref, be_ref)
    h3r = jnp.maximum(y3_ref[...].astype(jnp.float32) * scale + shift, 0.0)
    out_ref[...] = (jnp.sum(h3r * ow_ref[...], axis=1, keepdims=True)
                    + lo3_ref[...] + ob_ref[...])


def _const_spec(shape):
    nd = len(shape)
    return pl.BlockSpec(shape, lambda i: (0,) * nd)


def _tile_spec(cols, rows=_TILE):
    return pl.BlockSpec((rows, cols), lambda i: (i, 0))


_SEQ = pltpu.CompilerParams(dimension_semantics=("arbitrary",))


def kernel(user_id, gender, age, occupation, movie_id, genres, rating, implicit,
           emb_user, emb_gender, emb_age, emb_occ, emb_movie,
           genre_W, genre_b, ug_W, ug_b,
           b1_W, b1_b, b1_g, b1_beta, b1_pW, b1_pb,
           b2_W, b2_b, b2_g, b2_beta,
           b3_W, b3_b, b3_g, b3_beta, b3_pW, b3_pb,
           out_W, out_b):
    f32 = jnp.float32
    bf16 = jnp.bfloat16
    idx3 = jnp.concatenate([
        user_id.astype(jnp.int32),
        movie_id.astype(jnp.int32),
        (gender * 256 + age * 32 + occupation).astype(jnp.int32),
    ])

    big = jnp.concatenate([emb_user, emb_movie], axis=1)
    g1024 = jnp.arange(1024)
    combo = jnp.concatenate(
        [emb_gender[g1024 // 256], emb_age[(g1024 // 32) % 8],
         emb_occ[g1024 % 32], jnp.zeros((1024, 96), f32)], axis=1)

    u128, m128, c128 = _sc_gather3(idx3, big, combo)

    T = ug_W.reshape(64, 1024).astype(bf16)
    c1024 = jnp.arange(1024)
    Qw = jnp.pad((jnp.arange(32)[:, None] == (c1024[None, :] // 32)
                  ).astype(bf16), ((32, 64), (0, 0)))
    Pw = jnp.pad(((c1024[:, None] % 32) == jnp.arange(32)[None, :]
                  ).astype(bf16), ((0, 0), (64, 32)))
    GWw = jnp.pad(genre_W, ((0, 0), (32, 64)))
    gbw = jnp.pad(genre_b[None, :], ((0, 0), (32, 64)))
    ugbw = jnp.pad(ug_b[None, :], ((0, 0), (64, 32)))

    def permute_w1(W):
        return jnp.concatenate(
            [W[0:64], W[96:160], W[64:96], W[160:192], W[194:226],
             W[192:193], W[192:193], W[193:194],
             jnp.zeros((29, W.shape[1]), f32)], axis=0)

    W1b = permute_w1(b1_W).astype(bf16)
    pW1b = permute_w1(b1_pW).astype(bf16)

    x16, s1, q1 = pl.pallas_call(
        _k1,
        grid=(_GRID,),
        in_specs=[
            _tile_spec(128), _tile_spec(128), _tile_spec(128),
            _tile_spec(19), _tile_spec(1), _tile_spec(1),
            _const_spec((64, 1024)), _const_spec((128, 1024)),
            _const_spec((1024, 128)), _const_spec((19, 128)),
            _const_spec((1, 128)), _const_spec((1, 128)),
            _const_spec((_F, 1024)),
        ],
        out_specs=[
            _tile_spec(_F),
            _const_spec((1, 1024)), _const_spec((1, 1024)),
        ],
        out_shape=[
            jax.ShapeDtypeStruct((_B, _F), bf16),
            jax.ShapeDtypeStruct((1, 1024), f32),
            jax.ShapeDtypeStruct((1, 1024), f32),
        ],
        compiler_params=_SEQ,
    )(u128, m128, c128, genres, rating[:, None], implicit[:, None],
      T, Qw, Pw, GWw, gbw, ugbw, W1b)

    h1, y2, s2, q2 = pl.pallas_call(
        _k2,
        grid=(_GRID,),
        in_specs=[
            _tile_spec(_F),
            _const_spec((1, 1024)), _const_spec((1, 1024)),
            _const_spec((1, 1024)), _const_spec((1, 1024)),
            _const_spec((_F, 1024)), _const_spec((_F, 1024)),
            _const_spec((1, 1024)), _const_spec((1024, 1024)),
        ],
        out_specs=[
            _tile_spec(1024), _tile_spec(1024),
            _const_spec((1, 1024)), _const_spec((1, 1024)),
        ],
        out_shape=[
            jax.ShapeDtypeStruct((_B, 1024), bf16),
            jax.ShapeDtypeStruct((_B, 1024), bf16),
            jax.ShapeDtypeStruct((1, 1024), f32),
            jax.ShapeDtypeStruct((1, 1024), f32),
        ],
        scratch_shapes=[pltpu.VMEM((1024, 1024), bf16)],
        compiler_params=_SEQ,
    )(x16, s1, q1, b1_g.reshape(1, 1024), b1_beta.reshape(1, 1024),
      W1b, pW1b, b1_pb.reshape(1, 1024), b2_W)

    ow = out_W.reshape(1, 512)
    t3 = 1024
    y3, lo3, s3, q3 = pl.pallas_call(
        _k3,
        grid=(_B // t3,),
        in_specs=[
            _tile_spec(1024, t3), _tile_spec(1024, t3),
            _const_spec((1, 1024)), _const_spec((1, 1024)),
            _const_spec((1, 1024)), _const_spec((1, 1024)),
            _const_spec((1024, 512)), _const_spec((1024, 512)),
            _const_spec((1, 512)), _const_spec((1, 512)),
        ],
        out_specs=[
            _tile_spec(512, t3), _tile_spec(1, t3),
            _const_spec((1, 512)), _const_spec((1, 512)),
        ],
        out_shape=[
            jax.ShapeDtypeStruct((_B, 512), bf16),
            jax.ShapeDtypeStruct((_B, 1), f32),
            jax.ShapeDtypeStruct((1, 512), f32),
            jax.ShapeDtypeStruct((1, 512), f32),
        ],
        scratch_shapes=[pltpu.VMEM((1024, 512), bf16),
                        pltpu.VMEM((1024, 512), bf16)],
        compiler_params=_SEQ,
    )(h1, y2, s2, q2, b2_g.reshape(1, 1024), b2_beta.reshape(1, 1024),
      b3_W, b3_pW, b3_pb.reshape(1, 512), ow)

    t4 = 2048
    out = pl.pallas_call(
        _k4,
        grid=(_B // t4,),
        in_specs=[
            _tile_spec(512, t4), _tile_spec(1, t4),
            _const_spec((1, 512)), _const_spec((1, 512)),
            _const_spec((1, 512)), _const_spec((1, 512)),
            _const_spec((1, 512)), _const_spec((1, 1)),
        ],
        out_specs=_tile_spec(1, t4),
        out_shape=jax.ShapeDtypeStruct((_B, 1), f32),
        compiler_params=_SEQ,
    )(y3, lo3, s3, q3, b3_g.reshape(1, 512), b3_beta.reshape(1, 512),
      ow, out_b.reshape(1, 1))

    return out[:, 0]
